# baseline (speedup 1.0000x reference)
import functools

import jax
import jax.numpy as jnp
from jax import lax
from jax.experimental import pallas as pl
from jax.experimental.pallas import tpu as pltpu
from jax.experimental.pallas import tpu_sc as plsc

N_NODES = 10000
NFEAT = 128
N_EDGES = 320000

NC = 2
NS = 16
LANES = 16

CH = 128
SUB = 1
UNIT = CH * SUB
UPT = 160
E_PAD = UNIT * UPT * NS
IB = 4
WR = UNIT // 8
CROWS = 2
R_MAIN = 624
TAIL0 = NS * R_MAIN
TAIL = N_NODES - TAIL0

MTILE = 400
GRID = N_NODES // MTILE


def _sc_conv_body(x0_hbm, hwa_hbm, hwb_hbm,
                  idx1_hbm, wx1_hbm, idx2_hbm, wx2_hbm,
                  out0_hbm, out1_hbm,
                  accum, idx_v0, idx_v1, idx_v2, idx_v3,
                  wexp_v0, wexp_v1, wexp_v2, wexp_v3,
                  rows_v0, rows_v1,
                  gsem0, gsem1, ssem0, ssem1,
                  isem0, isem1, isem2, isem3,
                  wsem0, wsem1, wsem2, wsem3):
    idx_v = (idx_v0, idx_v1, idx_v2, idx_v3)
    wexp_v = (wexp_v0, wexp_v1, wexp_v2, wexp_v3)
    rows_v = (rows_v0, rows_v1)
    gsem = (gsem0, gsem1)
    ssem = (ssem0, ssem1)
    isem = (isem0, isem1, isem2, isem3)
    wsem = (wsem0, wsem1, wsem2, wsem3)
    c = lax.axis_index("c")
    s = lax.axis_index("s")
    row0 = s * R_MAIN
    last = s == NS - 1

    @pl.when(c == 0)
    def _():
        pltpu.sync_copy(x0_hbm.at[pl.ds(row0, R_MAIN)],
                        accum.at[pl.ds(row0, R_MAIN)])

        @pl.when(last)
        def _():
            pltpu.sync_copy(x0_hbm.at[pl.ds(TAIL0, TAIL)],
                            accum.at[pl.ds(TAIL0, TAIL)])

    @pl.when(c == 1)
    def _():
        def zrow(r, carry):
            for k in range(NFEAT // LANES):
                rows_v0[r, pl.ds(k * LANES, LANES)] = jnp.zeros(
                    (LANES,), jnp.float32)
            return carry
        lax.fori_loop(0, UNIT, zrow, 0)

        for j in range(R_MAIN // UNIT):
            pltpu.sync_copy(rows_v0,
                            accum.at[pl.ds(row0 + j * UNIT, UNIT)])
        rem = R_MAIN % UNIT
        pltpu.sync_copy(
            rows_v0.at[pl.ds(0, rem)],
            accum.at[pl.ds(row0 + (R_MAIN // UNIT) * UNIT, rem)])

        @pl.when(last)
        def _():
            pltpu.sync_copy(rows_v0.at[pl.ds(0, TAIL)],
                            accum.at[pl.ds(TAIL0, TAIL)])

    plsc.subcore_barrier()

    def edge_loop(hw_hbm, idx_hbm, wx_hbm):
        start = s * UPT

        def start_idx(j, q):
            pltpu.async_copy(idx_hbm.at[start + j], idx_v[q], isem[q])
            pltpu.async_copy(wx_hbm.at[start + j], wexp_v[q], wsem[q])

        def wait_idx(q):
            pltpu.make_async_copy(idx_hbm.at[start], idx_v[q],
                                  isem[q]).wait()

        def wait_wexp(q):
            pltpu.make_async_copy(wx_hbm.at[start], wexp_v[q],
                                  wsem[q]).wait()

        def start_gather(b, q):
            pltpu.async_copy(hw_hbm.at[idx_v[q].at[0]], rows_v[b],
                             gsem[b])

        def wait_gather(b, q):
            pltpu.make_async_copy(hw_hbm.at[idx_v[q].at[0]], rows_v[b],
                                  gsem[b]).wait()

        def scale(b, q):
            wait_wexp(q)

            def grp(r, gcarry):
                for ii in range(8):
                    e = r * 8 + ii
                    w = wexp_v[q][r, pl.ds(ii * LANES, LANES)]
                    for k in range(NFEAT // LANES):
                        sl = pl.ds(k * LANES, LANES)
                        rows_v[b][e, sl] = rows_v[b][e, sl] * w
                return gcarry
            lax.fori_loop(0, WR, grp, 0, unroll=2)

        def start_scatter(b, q):
            pltpu.async_copy(rows_v[b], accum.at[idx_v[q].at[1]],
                             ssem[b], add=True)

        def wait_scatter(b, q):
            pltpu.make_async_copy(rows_v[b], accum.at[idx_v[q].at[1]],
                                  ssem[b]).wait()

        start_idx(0, 0)
        start_idx(1, 1)
        wait_idx(0)
        start_gather(0, 0)

        def quad(p, carry):
            for ii in range(IB):
                i = p * IB + ii
                b = ii % 2
                o = 1 - b
                q = ii
                if ii == 0:
                    @pl.when(p > 0)
                    def _():
                        wait_scatter(o, (ii - 1) % IB)
                else:
                    wait_scatter(o, (ii - 1) % IB)
                j2 = jnp.where(i + 2 >= UPT, i + 2 - UPT, i + 2)
                start_idx(j2, (ii + 2) % IB)
                wait_idx((ii + 1) % IB)
                start_gather(o, (ii + 1) % IB)
                wait_gather(b, q)
                scale(b, q)
                start_scatter(b, q)
            return carry
        lax.fori_loop(0, UPT // IB, quad, 0)

        wait_scatter(1, (UPT - 1) % IB)
        wait_gather(0, UPT % IB)
        wait_idx(1)
        wait_wexp(0)
        wait_wexp(1)

    @pl.when(c == 0)
    def _():
        edge_loop(hwa_hbm, idx1_hbm, wx1_hbm)

    @pl.when(c == 1)
    def _():
        edge_loop(hwb_hbm, idx2_hbm, wx2_hbm)

    plsc.subcore_barrier()

    def writeout(out_hbm):
        pltpu.sync_copy(accum.at[pl.ds(row0, R_MAIN)],
                        out_hbm.at[pl.ds(row0, R_MAIN)])

        @pl.when(last)
        def _():
            pltpu.sync_copy(accum.at[pl.ds(TAIL0, TAIL)],
                            out_hbm.at[pl.ds(TAIL0, TAIL)])

    @pl.when(c == 0)
    def _():
        writeout(out0_hbm)

    @pl.when(c == 1)
    def _():
        writeout(out1_hbm)


_sc_conv = pl.kernel(
    _sc_conv_body,
    out_type=(jax.ShapeDtypeStruct((N_NODES, NFEAT), jnp.float32),
              jax.ShapeDtypeStruct((N_NODES, NFEAT), jnp.float32)),
    mesh=plsc.VectorSubcoreMesh(core_axis_name="c", subcore_axis_name="s"),
    scratch_types=(
        [pltpu.VMEM_SHARED((N_NODES, NFEAT), jnp.float32)]
        + [pltpu.VMEM((CROWS, NFEAT), jnp.int32)] * IB
        + [pltpu.VMEM((WR, NFEAT), jnp.float32)] * IB
        + [pltpu.VMEM((UNIT, NFEAT), jnp.float32)] * 2
        + [pltpu.SemaphoreType.DMA] * (4 + 2 * IB)
    ),
)


def _mm_body(two_prev, *refs):
    if two_prev:
        p0, p1, lnW, Wa, Wb, bsum, x0, hwa, hwb = refs
        h = p0[...] + p1[...]
    else:
        p0, lnW, Wa, Wb, bsum, x0, hwa, hwb = refs
        h = p0[...]
    x0[...] = jnp.dot(h, lnW[...], preferred_element_type=jnp.float32) + bsum[...]
    hwa[...] = jnp.dot(h, Wa[...], preferred_element_type=jnp.float32)
    hwb[...] = jnp.dot(h, Wb[...], preferred_element_type=jnp.float32)


def _make_mm(two_prev):
    n_prev = 2 if two_prev else 1
    in_specs = [pl.BlockSpec((MTILE, NFEAT), lambda i: (i, 0))
                for _ in range(n_prev)]
    in_specs += [pl.BlockSpec((NFEAT, NFEAT), lambda i: (0, 0))
                 for _ in range(3)]
    in_specs += [pl.BlockSpec((1, NFEAT), lambda i: (0, 0))]
    out_specs = [pl.BlockSpec((MTILE, NFEAT), lambda i: (i, 0))
                 for _ in range(3)]
    return pl.pallas_call(
        functools.partial(_mm_body, two_prev),
        grid=(GRID,),
        in_specs=in_specs,
        out_specs=out_specs,
        out_shape=[jax.ShapeDtypeStruct((N_NODES, NFEAT), jnp.float32)] * 3,
    )


_mm_one = _make_mm(False)
_mm_two = _make_mm(True)


def _add_body(a, b, o):
    o[...] = a[...] + b[...]


_combine = pl.pallas_call(
    _add_body,
    grid=(GRID,),
    in_specs=[pl.BlockSpec((MTILE, NFEAT), lambda i: (i, 0))] * 2,
    out_specs=pl.BlockSpec((MTILE, NFEAT), lambda i: (i, 0)),
    out_shape=jax.ShapeDtypeStruct((N_NODES, NFEAT), jnp.float32),
)


def kernel(x, edge_index, edge_attr, edge_index2, edge_attr2, batch,
           ln1_W, ln1_b, c1a_W, c1a_b, c1b_W, c1b_b,
           ln2_W, ln2_b, c2a_W, c2a_b, c2b_W, c2b_b,
           ln3_W, ln3_b, c3a_W, c3a_b, c3b_W, c3b_b):
    pad_i = jnp.arange(E_PAD - N_EDGES, dtype=jnp.int32) % N_NODES
    pad_f = jnp.zeros((E_PAD - N_EDGES,), jnp.float32)
    nun = E_PAD // UNIT

    def build_idx(ei):
        ei = ei.astype(jnp.int32)
        src = jnp.concatenate([ei[0], pad_i]).reshape(nun, 1, NFEAT)
        dst = jnp.concatenate([ei[1], pad_i]).reshape(nun, 1, NFEAT)
        return jnp.concatenate([src, dst], axis=1)

    def build_wexp(ea):
        eap = jnp.concatenate([ea.astype(jnp.float32), pad_f])
        eax = jnp.broadcast_to(eap[:, None], (E_PAD, LANES))
        return eax.reshape(nun, WR, NFEAT)

    idx1 = build_idx(edge_index)
    idx2 = build_idx(edge_index2)
    wx1 = build_wexp(edge_attr)
    wx2 = build_wexp(edge_attr2)

    params = [
        (ln1_W, ln1_b, c1a_W, c1a_b, c1b_W, c1b_b),
        (ln2_W, ln2_b, c2a_W, c2a_b, c2b_W, c2b_b),
        (ln3_W, ln3_b, c3a_W, c3a_b, c3b_W, c3b_b),
    ]

    prev = (x,)
    for lnW, lnb, Wa, ba, Wb, bb in params:
        bsum = (lnb + ba + bb).reshape(1, NFEAT)
        mm = _mm_one if len(prev) == 1 else _mm_two
        x0, hwa, hwb = mm(*prev, lnW, Wa, Wb, bsum)
        out0, out1 = _sc_conv(x0, hwa, hwb, idx1, wx1, idx2, wx2)
        prev = (out0, out1)

    return _combine(*prev)


# speedup vs baseline: 1.1041x; 1.1041x over previous
import functools

import jax
import jax.numpy as jnp
from jax import lax
from jax.experimental import pallas as pl
from jax.experimental.pallas import tpu as pltpu
from jax.experimental.pallas import tpu_sc as plsc

N_NODES = 10000
NFEAT = 128
N_EDGES = 320000

NC = 2
NS = 16
LANES = 16

CH = 128
SUB = 1
UNIT = CH * SUB
UPT = 160
E_PAD = UNIT * UPT * NS
IB = 4
WR = UNIT // 8
CROWS = 2
R_MAIN = 624
TAIL0 = NS * R_MAIN
TAIL = N_NODES - TAIL0

MTILE = 400
GRID = N_NODES // MTILE


def _sc_conv_body(x0_hbm, hwa_hbm, hwb_hbm,
                  idx1_hbm, wx1_hbm, idx2_hbm, wx2_hbm,
                  out0_hbm, out1_hbm,
                  accum, idx_v0, idx_v1, idx_v2, idx_v3,
                  wexp_v0, wexp_v1, wexp_v2, wexp_v3,
                  rows_v0, rows_v1,
                  gsem0, gsem1, ssem0, ssem1,
                  isem0, isem1, isem2, isem3,
                  wsem0, wsem1, wsem2, wsem3):
    idx_v = (idx_v0, idx_v1, idx_v2, idx_v3)
    wexp_v = (wexp_v0, wexp_v1, wexp_v2, wexp_v3)
    rows_v = (rows_v0, rows_v1)
    gsem = (gsem0, gsem1)
    ssem = (ssem0, ssem1)
    isem = (isem0, isem1, isem2, isem3)
    wsem = (wsem0, wsem1, wsem2, wsem3)
    c = lax.axis_index("c")
    s = lax.axis_index("s")
    row0 = s * R_MAIN
    last = s == NS - 1

    @pl.when(c == 0)
    def _():
        pltpu.sync_copy(x0_hbm.at[pl.ds(row0, R_MAIN)],
                        accum.at[pl.ds(row0, R_MAIN)])

        @pl.when(last)
        def _():
            pltpu.sync_copy(x0_hbm.at[pl.ds(TAIL0, TAIL)],
                            accum.at[pl.ds(TAIL0, TAIL)])

    @pl.when(c == 1)
    def _():
        def zrow(r, carry):
            for k in range(NFEAT // LANES):
                rows_v0[r, pl.ds(k * LANES, LANES)] = jnp.zeros(
                    (LANES,), jnp.float32)
            return carry
        lax.fori_loop(0, UNIT, zrow, 0)

        for j in range(R_MAIN // UNIT):
            pltpu.sync_copy(rows_v0,
                            accum.at[pl.ds(row0 + j * UNIT, UNIT)])
        rem = R_MAIN % UNIT
        pltpu.sync_copy(
            rows_v0.at[pl.ds(0, rem)],
            accum.at[pl.ds(row0 + (R_MAIN // UNIT) * UNIT, rem)])

        @pl.when(last)
        def _():
            pltpu.sync_copy(rows_v0.at[pl.ds(0, TAIL)],
                            accum.at[pl.ds(TAIL0, TAIL)])

    plsc.subcore_barrier()

    def edge_loop(hw_hbm, idx_hbm, wx_hbm):
        start = s * UPT

        def start_idx(j, q):
            pltpu.async_copy(idx_hbm.at[start + j], idx_v[q], isem[q])
            pltpu.async_copy(wx_hbm.at[start + j], wexp_v[q], wsem[q])

        def wait_idx(q):
            pltpu.make_async_copy(idx_hbm.at[start], idx_v[q],
                                  isem[q]).wait()

        def wait_wexp(q):
            pltpu.make_async_copy(wx_hbm.at[start], wexp_v[q],
                                  wsem[q]).wait()

        def start_gather(b, q):
            pltpu.async_copy(hw_hbm.at[idx_v[q].at[0]], rows_v[b],
                             gsem[b])

        def wait_gather(b, q):
            pltpu.make_async_copy(hw_hbm.at[idx_v[q].at[0]], rows_v[b],
                                  gsem[b]).wait()

        def scale(b, q):
            wait_wexp(q)

            nk = NFEAT // LANES

            def grp(r, gcarry):
                for half in range(2):
                    ids = [half * 4 + t for t in range(4)]
                    ws = [wexp_v[q][r, pl.ds(ii * LANES, LANES)]
                          for ii in ids]
                    loaded = [[rows_v[b][r * 8 + ii, pl.ds(k * LANES,
                                                           LANES)]
                               for k in range(nk)] for ii in ids]
                    for t, ii in enumerate(ids):
                        for k in range(nk):
                            rows_v[b][r * 8 + ii,
                                      pl.ds(k * LANES, LANES)] = (
                                loaded[t][k] * ws[t])
                return gcarry
            lax.fori_loop(0, WR, grp, 0)

        def start_scatter(b, q):
            pltpu.async_copy(rows_v[b], accum.at[idx_v[q].at[1]],
                             ssem[b], add=True)

        def wait_scatter(b, q):
            pltpu.make_async_copy(rows_v[b], accum.at[idx_v[q].at[1]],
                                  ssem[b]).wait()

        start_idx(0, 0)
        start_idx(1, 1)
        wait_idx(0)
        start_gather(0, 0)

        def quad(p, carry):
            for ii in range(IB):
                i = p * IB + ii
                b = ii % 2
                o = 1 - b
                q = ii
                if ii == 0:
                    @pl.when(p > 0)
                    def _():
                        wait_scatter(o, (ii - 1) % IB)
                else:
                    wait_scatter(o, (ii - 1) % IB)
                j2 = jnp.where(i + 2 >= UPT, i + 2 - UPT, i + 2)
                start_idx(j2, (ii + 2) % IB)
                wait_idx((ii + 1) % IB)
                start_gather(o, (ii + 1) % IB)
                wait_gather(b, q)
                scale(b, q)
                start_scatter(b, q)
            return carry
        lax.fori_loop(0, UPT // IB, quad, 0)

        wait_scatter(1, (UPT - 1) % IB)
        wait_gather(0, UPT % IB)
        wait_idx(1)
        wait_wexp(0)
        wait_wexp(1)

    @pl.when(c == 0)
    def _():
        edge_loop(hwa_hbm, idx1_hbm, wx1_hbm)

    @pl.when(c == 1)
    def _():
        edge_loop(hwb_hbm, idx2_hbm, wx2_hbm)

    plsc.subcore_barrier()

    def writeout(out_hbm):
        pltpu.sync_copy(accum.at[pl.ds(row0, R_MAIN)],
                        out_hbm.at[pl.ds(row0, R_MAIN)])

        @pl.when(last)
        def _():
            pltpu.sync_copy(accum.at[pl.ds(TAIL0, TAIL)],
                            out_hbm.at[pl.ds(TAIL0, TAIL)])

    @pl.when(c == 0)
    def _():
        writeout(out0_hbm)

    @pl.when(c == 1)
    def _():
        writeout(out1_hbm)


_sc_conv = pl.kernel(
    _sc_conv_body,
    out_type=(jax.ShapeDtypeStruct((N_NODES, NFEAT), jnp.float32),
              jax.ShapeDtypeStruct((N_NODES, NFEAT), jnp.float32)),
    mesh=plsc.VectorSubcoreMesh(core_axis_name="c", subcore_axis_name="s"),
    scratch_types=(
        [pltpu.VMEM_SHARED((N_NODES, NFEAT), jnp.float32)]
        + [pltpu.VMEM((CROWS, NFEAT), jnp.int32)] * IB
        + [pltpu.VMEM((WR, NFEAT), jnp.float32)] * IB
        + [pltpu.VMEM((UNIT, NFEAT), jnp.float32)] * 2
        + [pltpu.SemaphoreType.DMA] * (4 + 2 * IB)
    ),
)


def _mm_body(two_prev, *refs):
    if two_prev:
        p0, p1, lnW, Wa, Wb, bsum, x0, hwa, hwb = refs
        h = p0[...] + p1[...]
    else:
        p0, lnW, Wa, Wb, bsum, x0, hwa, hwb = refs
        h = p0[...]
    x0[...] = jnp.dot(h, lnW[...], preferred_element_type=jnp.float32) + bsum[...]
    hwa[...] = jnp.dot(h, Wa[...], preferred_element_type=jnp.float32)
    hwb[...] = jnp.dot(h, Wb[...], preferred_element_type=jnp.float32)


def _make_mm(two_prev):
    n_prev = 2 if two_prev else 1
    in_specs = [pl.BlockSpec((MTILE, NFEAT), lambda i: (i, 0))
                for _ in range(n_prev)]
    in_specs += [pl.BlockSpec((NFEAT, NFEAT), lambda i: (0, 0))
                 for _ in range(3)]
    in_specs += [pl.BlockSpec((1, NFEAT), lambda i: (0, 0))]
    out_specs = [pl.BlockSpec((MTILE, NFEAT), lambda i: (i, 0))
                 for _ in range(3)]
    return pl.pallas_call(
        functools.partial(_mm_body, two_prev),
        grid=(GRID,),
        in_specs=in_specs,
        out_specs=out_specs,
        out_shape=[jax.ShapeDtypeStruct((N_NODES, NFEAT), jnp.float32)] * 3,
    )


_mm_one = _make_mm(False)
_mm_two = _make_mm(True)


def _add_body(a, b, o):
    o[...] = a[...] + b[...]


_combine = pl.pallas_call(
    _add_body,
    grid=(GRID,),
    in_specs=[pl.BlockSpec((MTILE, NFEAT), lambda i: (i, 0))] * 2,
    out_specs=pl.BlockSpec((MTILE, NFEAT), lambda i: (i, 0)),
    out_shape=jax.ShapeDtypeStruct((N_NODES, NFEAT), jnp.float32),
)


def kernel(x, edge_index, edge_attr, edge_index2, edge_attr2, batch,
           ln1_W, ln1_b, c1a_W, c1a_b, c1b_W, c1b_b,
           ln2_W, ln2_b, c2a_W, c2a_b, c2b_W, c2b_b,
           ln3_W, ln3_b, c3a_W, c3a_b, c3b_W, c3b_b):
    pad_i = jnp.arange(E_PAD - N_EDGES, dtype=jnp.int32) % N_NODES
    pad_f = jnp.zeros((E_PAD - N_EDGES,), jnp.float32)
    nun = E_PAD // UNIT

    def build_idx(ei):
        ei = ei.astype(jnp.int32)
        src = jnp.concatenate([ei[0], pad_i]).reshape(nun, 1, NFEAT)
        dst = jnp.concatenate([ei[1], pad_i]).reshape(nun, 1, NFEAT)
        return jnp.concatenate([src, dst], axis=1)

    def build_wexp(ea):
        eap = jnp.concatenate([ea.astype(jnp.float32), pad_f])
        eax = jnp.broadcast_to(eap[:, None], (E_PAD, LANES))
        return eax.reshape(nun, WR, NFEAT)

    idx1 = build_idx(edge_index)
    idx2 = build_idx(edge_index2)
    wx1 = build_wexp(edge_attr)
    wx2 = build_wexp(edge_attr2)

    params = [
        (ln1_W, ln1_b, c1a_W, c1a_b, c1b_W, c1b_b),
        (ln2_W, ln2_b, c2a_W, c2a_b, c2b_W, c2b_b),
        (ln3_W, ln3_b, c3a_W, c3a_b, c3b_W, c3b_b),
    ]

    prev = (x,)
    for lnW, lnb, Wa, ba, Wb, bb in params:
        bsum = (lnb + ba + bb).reshape(1, NFEAT)
        mm = _mm_one if len(prev) == 1 else _mm_two
        x0, hwa, hwb = mm(*prev, lnW, Wa, Wb, bsum)
        out0, out1 = _sc_conv(x0, hwa, hwb, idx1, wx1, idx2, wx2)
        prev = (out0, out1)

    return _combine(*prev)


# speedup vs baseline: 1.1072x; 1.0028x over previous
import functools

import jax
import jax.numpy as jnp
from jax import lax
from jax.experimental import pallas as pl
from jax.experimental.pallas import tpu as pltpu
from jax.experimental.pallas import tpu_sc as plsc

N_NODES = 10000
NFEAT = 128
N_EDGES = 320000

NC = 2
NS = 16
LANES = 16

CH = 128
SUB = 1
UNIT = CH * SUB
UPT = 160
E_PAD = UNIT * UPT * NS
IB = 4
WR = UNIT // 8
CROWS = 2
R_MAIN = 624
TAIL0 = NS * R_MAIN
TAIL = N_NODES - TAIL0

MTILE = 400
GRID = N_NODES // MTILE


def _sc_conv_body(x0_hbm, hwa_hbm, hwb_hbm,
                  idx1_hbm, wx1_hbm, idx2_hbm, wx2_hbm,
                  out0_hbm, out1_hbm,
                  accum, idx_v0, idx_v1, idx_v2, idx_v3,
                  wexp_v0, wexp_v1, wexp_v2, wexp_v3,
                  rows_v0, rows_v1,
                  gsem0, gsem1, ssem0, ssem1,
                  isem0, isem1, isem2, isem3,
                  wsem0, wsem1, wsem2, wsem3):
    idx_v = (idx_v0, idx_v1, idx_v2, idx_v3)
    wexp_v = (wexp_v0, wexp_v1, wexp_v2, wexp_v3)
    rows_v = (rows_v0, rows_v1)
    gsem = (gsem0, gsem1)
    ssem = (ssem0, ssem1)
    isem = (isem0, isem1, isem2, isem3)
    wsem = (wsem0, wsem1, wsem2, wsem3)
    c = lax.axis_index("c")
    s = lax.axis_index("s")
    row0 = s * R_MAIN
    last = s == NS - 1

    @pl.when(c == 0)
    def _():
        pltpu.sync_copy(x0_hbm.at[pl.ds(row0, R_MAIN)],
                        accum.at[pl.ds(row0, R_MAIN)])

        @pl.when(last)
        def _():
            pltpu.sync_copy(x0_hbm.at[pl.ds(TAIL0, TAIL)],
                            accum.at[pl.ds(TAIL0, TAIL)])

    @pl.when(c == 1)
    def _():
        def zrow(r, carry):
            for k in range(NFEAT // LANES):
                rows_v0[r, pl.ds(k * LANES, LANES)] = jnp.zeros(
                    (LANES,), jnp.float32)
            return carry
        lax.fori_loop(0, UNIT, zrow, 0)

        for j in range(R_MAIN // UNIT):
            pltpu.sync_copy(rows_v0,
                            accum.at[pl.ds(row0 + j * UNIT, UNIT)])
        rem = R_MAIN % UNIT
        pltpu.sync_copy(
            rows_v0.at[pl.ds(0, rem)],
            accum.at[pl.ds(row0 + (R_MAIN // UNIT) * UNIT, rem)])

        @pl.when(last)
        def _():
            pltpu.sync_copy(rows_v0.at[pl.ds(0, TAIL)],
                            accum.at[pl.ds(TAIL0, TAIL)])

    plsc.subcore_barrier()

    def edge_loop(hw_hbm, idx_hbm, wx_hbm):
        start = s * UPT

        def start_idx(j, q):
            pltpu.async_copy(idx_hbm.at[start + j], idx_v[q], isem[q])
            pltpu.async_copy(wx_hbm.at[start + j], wexp_v[q], wsem[q])

        def wait_idx(q):
            pltpu.make_async_copy(idx_hbm.at[start], idx_v[q],
                                  isem[q]).wait()

        def wait_wexp(q):
            pltpu.make_async_copy(wx_hbm.at[start], wexp_v[q],
                                  wsem[q]).wait()

        def start_gather(b, q):
            pltpu.async_copy(hw_hbm.at[idx_v[q].at[0]], rows_v[b],
                             gsem[b])

        def wait_gather(b, q):
            pltpu.make_async_copy(hw_hbm.at[idx_v[q].at[0]], rows_v[b],
                                  gsem[b]).wait()

        def scale(b, q):
            wait_wexp(q)

            nk = NFEAT // LANES

            def grp(r, gcarry):
                for half in range(2):
                    ids = [half * 4 + t for t in range(4)]
                    ws = [wexp_v[q][r, pl.ds(ii * LANES, LANES)]
                          for ii in ids]
                    loaded = [[rows_v[b][r * 8 + ii, pl.ds(k * LANES,
                                                           LANES)]
                               for k in range(nk)] for ii in ids]
                    for t, ii in enumerate(ids):
                        for k in range(nk):
                            rows_v[b][r * 8 + ii,
                                      pl.ds(k * LANES, LANES)] = (
                                loaded[t][k] * ws[t])
                return gcarry
            lax.fori_loop(0, WR, grp, 0, unroll=2)

        def start_scatter(b, q):
            pltpu.async_copy(rows_v[b], accum.at[idx_v[q].at[1]],
                             ssem[b], add=True)

        def wait_scatter(b, q):
            pltpu.make_async_copy(rows_v[b], accum.at[idx_v[q].at[1]],
                                  ssem[b]).wait()

        start_idx(0, 0)
        start_idx(1, 1)
        wait_idx(0)
        start_gather(0, 0)

        def quad(p, carry):
            for ii in range(IB):
                i = p * IB + ii
                b = ii % 2
                o = 1 - b
                q = ii
                if ii == 0:
                    @pl.when(p > 0)
                    def _():
                        wait_scatter(o, (ii - 1) % IB)
                else:
                    wait_scatter(o, (ii - 1) % IB)
                j2 = jnp.where(i + 2 >= UPT, i + 2 - UPT, i + 2)
                start_idx(j2, (ii + 2) % IB)
                wait_idx((ii + 1) % IB)
                start_gather(o, (ii + 1) % IB)
                wait_gather(b, q)
                scale(b, q)
                start_scatter(b, q)
            return carry
        lax.fori_loop(0, UPT // IB, quad, 0)

        wait_scatter(1, (UPT - 1) % IB)
        wait_gather(0, UPT % IB)
        wait_idx(1)
        wait_wexp(0)
        wait_wexp(1)

    @pl.when(c == 0)
    def _():
        edge_loop(hwa_hbm, idx1_hbm, wx1_hbm)

    @pl.when(c == 1)
    def _():
        edge_loop(hwb_hbm, idx2_hbm, wx2_hbm)

    plsc.subcore_barrier()

    def writeout(out_hbm):
        pltpu.sync_copy(accum.at[pl.ds(row0, R_MAIN)],
                        out_hbm.at[pl.ds(row0, R_MAIN)])

        @pl.when(last)
        def _():
            pltpu.sync_copy(accum.at[pl.ds(TAIL0, TAIL)],
                            out_hbm.at[pl.ds(TAIL0, TAIL)])

    @pl.when(c == 0)
    def _():
        writeout(out0_hbm)

    @pl.when(c == 1)
    def _():
        writeout(out1_hbm)


_sc_conv = pl.kernel(
    _sc_conv_body,
    out_type=(jax.ShapeDtypeStruct((N_NODES, NFEAT), jnp.float32),
              jax.ShapeDtypeStruct((N_NODES, NFEAT), jnp.float32)),
    mesh=plsc.VectorSubcoreMesh(core_axis_name="c", subcore_axis_name="s"),
    scratch_types=(
        [pltpu.VMEM_SHARED((N_NODES, NFEAT), jnp.float32)]
        + [pltpu.VMEM((CROWS, NFEAT), jnp.int32)] * IB
        + [pltpu.VMEM((WR, NFEAT), jnp.float32)] * IB
        + [pltpu.VMEM((UNIT, NFEAT), jnp.float32)] * 2
        + [pltpu.SemaphoreType.DMA] * (4 + 2 * IB)
    ),
)


def _mm_body(two_prev, *refs):
    if two_prev:
        p0, p1, lnW, Wa, Wb, bsum, x0, hwa, hwb = refs
        h = p0[...] + p1[...]
    else:
        p0, lnW, Wa, Wb, bsum, x0, hwa, hwb = refs
        h = p0[...]
    x0[...] = jnp.dot(h, lnW[...], preferred_element_type=jnp.float32) + bsum[...]
    hwa[...] = jnp.dot(h, Wa[...], preferred_element_type=jnp.float32)
    hwb[...] = jnp.dot(h, Wb[...], preferred_element_type=jnp.float32)


def _make_mm(two_prev):
    n_prev = 2 if two_prev else 1
    in_specs = [pl.BlockSpec((MTILE, NFEAT), lambda i: (i, 0))
                for _ in range(n_prev)]
    in_specs += [pl.BlockSpec((NFEAT, NFEAT), lambda i: (0, 0))
                 for _ in range(3)]
    in_specs += [pl.BlockSpec((1, NFEAT), lambda i: (0, 0))]
    out_specs = [pl.BlockSpec((MTILE, NFEAT), lambda i: (i, 0))
                 for _ in range(3)]
    return pl.pallas_call(
        functools.partial(_mm_body, two_prev),
        grid=(GRID,),
        in_specs=in_specs,
        out_specs=out_specs,
        out_shape=[jax.ShapeDtypeStruct((N_NODES, NFEAT), jnp.float32)] * 3,
    )


_mm_one = _make_mm(False)
_mm_two = _make_mm(True)


def _add_body(a, b, o):
    o[...] = a[...] + b[...]


_combine = pl.pallas_call(
    _add_body,
    grid=(GRID,),
    in_specs=[pl.BlockSpec((MTILE, NFEAT), lambda i: (i, 0))] * 2,
    out_specs=pl.BlockSpec((MTILE, NFEAT), lambda i: (i, 0)),
    out_shape=jax.ShapeDtypeStruct((N_NODES, NFEAT), jnp.float32),
)


def kernel(x, edge_index, edge_attr, edge_index2, edge_attr2, batch,
           ln1_W, ln1_b, c1a_W, c1a_b, c1b_W, c1b_b,
           ln2_W, ln2_b, c2a_W, c2a_b, c2b_W, c2b_b,
           ln3_W, ln3_b, c3a_W, c3a_b, c3b_W, c3b_b):
    pad_i = jnp.arange(E_PAD - N_EDGES, dtype=jnp.int32) % N_NODES
    pad_f = jnp.zeros((E_PAD - N_EDGES,), jnp.float32)
    nun = E_PAD // UNIT

    def build_idx(ei):
        ei = ei.astype(jnp.int32)
        src = jnp.concatenate([ei[0], pad_i]).reshape(nun, 1, NFEAT)
        dst = jnp.concatenate([ei[1], pad_i]).reshape(nun, 1, NFEAT)
        return jnp.concatenate([src, dst], axis=1)

    def build_wexp(ea):
        eap = jnp.concatenate([ea.astype(jnp.float32), pad_f])
        eax = jnp.broadcast_to(eap[:, None], (E_PAD, LANES))
        return eax.reshape(nun, WR, NFEAT)

    idx1 = build_idx(edge_index)
    idx2 = build_idx(edge_index2)
    wx1 = build_wexp(edge_attr)
    wx2 = build_wexp(edge_attr2)

    params = [
        (ln1_W, ln1_b, c1a_W, c1a_b, c1b_W, c1b_b),
        (ln2_W, ln2_b, c2a_W, c2a_b, c2b_W, c2b_b),
        (ln3_W, ln3_b, c3a_W, c3a_b, c3b_W, c3b_b),
    ]

    prev = (x,)
    for lnW, lnb, Wa, ba, Wb, bb in params:
        bsum = (lnb + ba + bb).reshape(1, NFEAT)
        mm = _mm_one if len(prev) == 1 else _mm_two
        x0, hwa, hwb = mm(*prev, lnW, Wa, Wb, bsum)
        out0, out1 = _sc_conv(x0, hwa, hwb, idx1, wx1, idx2, wx2)
        prev = (out0, out1)

    return _combine(*prev)


# speedup vs baseline: 1.1398x; 1.0294x over previous
import functools

import jax
import jax.numpy as jnp
from jax import lax
from jax.experimental import pallas as pl
from jax.experimental.pallas import tpu as pltpu
from jax.experimental.pallas import tpu_sc as plsc

N_NODES = 10000
NFEAT = 128
N_EDGES = 320000

NC = 2
NS = 16
LANES = 16

CH = 128
SUB = 1
UNIT = CH * SUB
UPT = 160
E_PAD = UNIT * UPT * NS
IB = 4
WR = UNIT // 8
CROWS = 2
R_MAIN = 624
TAIL0 = NS * R_MAIN
TAIL = N_NODES - TAIL0

MTILE = 1000
GRID = N_NODES // MTILE


def _sc_conv_body(x0_hbm, hwa_hbm, hwb_hbm,
                  idx1_hbm, wx1_hbm, idx2_hbm, wx2_hbm,
                  out0_hbm, out1_hbm,
                  accum, idx_v0, idx_v1, idx_v2, idx_v3,
                  wexp_v0, wexp_v1, wexp_v2, wexp_v3,
                  rows_v0, rows_v1,
                  gsem0, gsem1, ssem0, ssem1,
                  isem0, isem1, isem2, isem3,
                  wsem0, wsem1, wsem2, wsem3):
    idx_v = (idx_v0, idx_v1, idx_v2, idx_v3)
    wexp_v = (wexp_v0, wexp_v1, wexp_v2, wexp_v3)
    rows_v = (rows_v0, rows_v1)
    gsem = (gsem0, gsem1)
    ssem = (ssem0, ssem1)
    isem = (isem0, isem1, isem2, isem3)
    wsem = (wsem0, wsem1, wsem2, wsem3)
    c = lax.axis_index("c")
    s = lax.axis_index("s")
    row0 = s * R_MAIN
    last = s == NS - 1

    @pl.when(c == 0)
    def _():
        pltpu.sync_copy(x0_hbm.at[pl.ds(row0, R_MAIN)],
                        accum.at[pl.ds(row0, R_MAIN)])

        @pl.when(last)
        def _():
            pltpu.sync_copy(x0_hbm.at[pl.ds(TAIL0, TAIL)],
                            accum.at[pl.ds(TAIL0, TAIL)])

    @pl.when(c == 1)
    def _():
        def zrow(r, carry):
            for k in range(NFEAT // LANES):
                rows_v0[r, pl.ds(k * LANES, LANES)] = jnp.zeros(
                    (LANES,), jnp.float32)
            return carry
        lax.fori_loop(0, UNIT, zrow, 0)

        for j in range(R_MAIN // UNIT):
            pltpu.sync_copy(rows_v0,
                            accum.at[pl.ds(row0 + j * UNIT, UNIT)])
        rem = R_MAIN % UNIT
        pltpu.sync_copy(
            rows_v0.at[pl.ds(0, rem)],
            accum.at[pl.ds(row0 + (R_MAIN // UNIT) * UNIT, rem)])

        @pl.when(last)
        def _():
            pltpu.sync_copy(rows_v0.at[pl.ds(0, TAIL)],
                            accum.at[pl.ds(TAIL0, TAIL)])

    plsc.subcore_barrier()

    def edge_loop(hw_hbm, idx_hbm, wx_hbm):
        start = s * UPT

        def start_idx(j, q):
            pltpu.async_copy(idx_hbm.at[start + j], idx_v[q], isem[q])
            pltpu.async_copy(wx_hbm.at[start + j], wexp_v[q], wsem[q])

        def wait_idx(q):
            pltpu.make_async_copy(idx_hbm.at[start], idx_v[q],
                                  isem[q]).wait()

        def wait_wexp(q):
            pltpu.make_async_copy(wx_hbm.at[start], wexp_v[q],
                                  wsem[q]).wait()

        def start_gather(b, q):
            pltpu.async_copy(hw_hbm.at[idx_v[q].at[0]], rows_v[b],
                             gsem[b])

        def wait_gather(b, q):
            pltpu.make_async_copy(hw_hbm.at[idx_v[q].at[0]], rows_v[b],
                                  gsem[b]).wait()

        def scale(b, q):
            wait_wexp(q)

            nk = NFEAT // LANES

            def grp(r, gcarry):
                for half in range(2):
                    ids = [half * 4 + t for t in range(4)]
                    ws = [wexp_v[q][r, pl.ds(ii * LANES, LANES)]
                          for ii in ids]
                    loaded = [[rows_v[b][r * 8 + ii, pl.ds(k * LANES,
                                                           LANES)]
                               for k in range(nk)] for ii in ids]
                    for t, ii in enumerate(ids):
                        for k in range(nk):
                            rows_v[b][r * 8 + ii,
                                      pl.ds(k * LANES, LANES)] = (
                                loaded[t][k] * ws[t])
                return gcarry
            lax.fori_loop(0, WR, grp, 0, unroll=2)

        def start_scatter(b, q):
            pltpu.async_copy(rows_v[b], accum.at[idx_v[q].at[1]],
                             ssem[b], add=True)

        def wait_scatter(b, q):
            pltpu.make_async_copy(rows_v[b], accum.at[idx_v[q].at[1]],
                                  ssem[b]).wait()

        start_idx(0, 0)
        start_idx(1, 1)
        wait_idx(0)
        start_gather(0, 0)

        def quad(p, carry):
            for ii in range(IB):
                i = p * IB + ii
                b = ii % 2
                o = 1 - b
                q = ii
                if ii == 0:
                    @pl.when(p > 0)
                    def _():
                        wait_scatter(o, (ii - 1) % IB)
                else:
                    wait_scatter(o, (ii - 1) % IB)
                j2 = jnp.where(i + 2 >= UPT, i + 2 - UPT, i + 2)
                start_idx(j2, (ii + 2) % IB)
                wait_idx((ii + 1) % IB)
                start_gather(o, (ii + 1) % IB)
                wait_gather(b, q)
                scale(b, q)
                start_scatter(b, q)
            return carry
        lax.fori_loop(0, UPT // IB, quad, 0)

        wait_scatter(1, (UPT - 1) % IB)
        wait_gather(0, UPT % IB)
        wait_idx(1)
        wait_wexp(0)
        wait_wexp(1)

    @pl.when(c == 0)
    def _():
        edge_loop(hwa_hbm, idx1_hbm, wx1_hbm)

    @pl.when(c == 1)
    def _():
        edge_loop(hwb_hbm, idx2_hbm, wx2_hbm)

    plsc.subcore_barrier()

    def writeout(out_hbm):
        pltpu.sync_copy(accum.at[pl.ds(row0, R_MAIN)],
                        out_hbm.at[pl.ds(row0, R_MAIN)])

        @pl.when(last)
        def _():
            pltpu.sync_copy(accum.at[pl.ds(TAIL0, TAIL)],
                            out_hbm.at[pl.ds(TAIL0, TAIL)])

    @pl.when(c == 0)
    def _():
        writeout(out0_hbm)

    @pl.when(c == 1)
    def _():
        writeout(out1_hbm)


_sc_conv = pl.kernel(
    _sc_conv_body,
    out_type=(jax.ShapeDtypeStruct((N_NODES, NFEAT), jnp.float32),
              jax.ShapeDtypeStruct((N_NODES, NFEAT), jnp.float32)),
    mesh=plsc.VectorSubcoreMesh(core_axis_name="c", subcore_axis_name="s"),
    scratch_types=(
        [pltpu.VMEM_SHARED((N_NODES, NFEAT), jnp.float32)]
        + [pltpu.VMEM((CROWS, NFEAT), jnp.int32)] * IB
        + [pltpu.VMEM((WR, NFEAT), jnp.float32)] * IB
        + [pltpu.VMEM((UNIT, NFEAT), jnp.float32)] * 2
        + [pltpu.SemaphoreType.DMA] * (4 + 2 * IB)
    ),
)


def _mm_body(two_prev, *refs):
    if two_prev:
        p0, p1, lnW, Wa, Wb, bsum, x0, hwa, hwb = refs
        h = p0[...] + p1[...]
    else:
        p0, lnW, Wa, Wb, bsum, x0, hwa, hwb = refs
        h = p0[...]
    x0[...] = jnp.dot(h, lnW[...], preferred_element_type=jnp.float32) + bsum[...]
    hwa[...] = jnp.dot(h, Wa[...], preferred_element_type=jnp.float32)
    hwb[...] = jnp.dot(h, Wb[...], preferred_element_type=jnp.float32)


def _make_mm(two_prev):
    n_prev = 2 if two_prev else 1
    in_specs = [pl.BlockSpec((MTILE, NFEAT), lambda i: (i, 0))
                for _ in range(n_prev)]
    in_specs += [pl.BlockSpec((NFEAT, NFEAT), lambda i: (0, 0))
                 for _ in range(3)]
    in_specs += [pl.BlockSpec((1, NFEAT), lambda i: (0, 0))]
    out_specs = [pl.BlockSpec((MTILE, NFEAT), lambda i: (i, 0))
                 for _ in range(3)]
    return pl.pallas_call(
        functools.partial(_mm_body, two_prev),
        grid=(GRID,),
        in_specs=in_specs,
        out_specs=out_specs,
        out_shape=[jax.ShapeDtypeStruct((N_NODES, NFEAT), jnp.float32)] * 3,
    )


_mm_one = _make_mm(False)
_mm_two = _make_mm(True)


def _add_body(a, b, o):
    o[...] = a[...] + b[...]


_combine = pl.pallas_call(
    _add_body,
    grid=(GRID,),
    in_specs=[pl.BlockSpec((MTILE, NFEAT), lambda i: (i, 0))] * 2,
    out_specs=pl.BlockSpec((MTILE, NFEAT), lambda i: (i, 0)),
    out_shape=jax.ShapeDtypeStruct((N_NODES, NFEAT), jnp.float32),
)


def kernel(x, edge_index, edge_attr, edge_index2, edge_attr2, batch,
           ln1_W, ln1_b, c1a_W, c1a_b, c1b_W, c1b_b,
           ln2_W, ln2_b, c2a_W, c2a_b, c2b_W, c2b_b,
           ln3_W, ln3_b, c3a_W, c3a_b, c3b_W, c3b_b):
    pad_i = jnp.arange(E_PAD - N_EDGES, dtype=jnp.int32) % N_NODES
    pad_f = jnp.zeros((E_PAD - N_EDGES,), jnp.float32)
    nun = E_PAD // UNIT

    def build_idx(ei):
        ei = ei.astype(jnp.int32)
        src = jnp.concatenate([ei[0], pad_i]).reshape(nun, 1, NFEAT)
        dst = jnp.concatenate([ei[1], pad_i]).reshape(nun, 1, NFEAT)
        return jnp.concatenate([src, dst], axis=1)

    def build_wexp(ea):
        eap = jnp.concatenate([ea.astype(jnp.float32), pad_f])
        eax = jnp.broadcast_to(eap[:, None], (E_PAD, LANES))
        return eax.reshape(nun, WR, NFEAT)

    idx1 = build_idx(edge_index)
    idx2 = build_idx(edge_index2)
    wx1 = build_wexp(edge_attr)
    wx2 = build_wexp(edge_attr2)

    params = [
        (ln1_W, ln1_b, c1a_W, c1a_b, c1b_W, c1b_b),
        (ln2_W, ln2_b, c2a_W, c2a_b, c2b_W, c2b_b),
        (ln3_W, ln3_b, c3a_W, c3a_b, c3b_W, c3b_b),
    ]

    prev = (x,)
    for lnW, lnb, Wa, ba, Wb, bb in params:
        bsum = (lnb + ba + bb).reshape(1, NFEAT)
        mm = _mm_one if len(prev) == 1 else _mm_two
        x0, hwa, hwb = mm(*prev, lnW, Wa, Wb, bsum)
        out0, out1 = _sc_conv(x0, hwa, hwb, idx1, wx1, idx2, wx2)
        prev = (out0, out1)

    return _combine(*prev)
